# baseline (device time: 536125 ns/iter reference)
import jax
import jax.numpy as jnp
from jax import lax
from jax.experimental import pallas as pl
from jax.experimental.pallas import tpu as pltpu

N_DEV = 16
B, Sq, Hq, Dh = 4, 256, 8, 128
D = Hq * Dh
SCALE = 0.08838834764831843
EXT = 128
W = D + EXT


def kernel(x, Wq, Wo, K_ext, V_ext):
    xb = x.astype(jnp.bfloat16)
    wqb = Wq.astype(jnp.bfloat16)
    wob = Wo.astype(jnp.bfloat16)
    kb = K_ext.astype(jnp.bfloat16)
    vb = V_ext.astype(jnp.bfloat16)

    def body(x_ref, wq_ref, wo_ref, k_ref, v_ref, out_ref,
             q_ref, acc_ref, comm_ref, attn_ref,
             send_sems, recv_sems, credit_sem):
        my = lax.axis_index("i")
        left = lax.rem(my - 1 + N_DEV, N_DEV)
        right = lax.rem(my + 1, N_DEV)

        barrier_sem = pltpu.get_barrier_semaphore()
        for nbr in (left, right):
            pl.semaphore_signal(barrier_sem, inc=1, device_id=(nbr,),
                                device_id_type=pl.DeviceIdType.MESH)
        pl.semaphore_wait(barrier_sem, 2)

        xm = x_ref[...].reshape(B * Sq, D)
        q_ref[...] = (lax.dot(xm, wq_ref[...],
                              preferred_element_type=jnp.float32)
                      * SCALE).astype(jnp.bfloat16)

        for b in range(B):
            r0 = b * Sq
            for h in range(Hq):
                c0 = h * Dh
                qbh = q_ref[r0:r0 + Sq, c0:c0 + Dh]
                kbh = k_ref[b, :, h, :]
                s = lax.dot_general(qbh, kbh, (((1,), (1,)), ((), ())),
                                    preferred_element_type=jnp.float32)
                p = jnp.exp(s)
                lvec = jnp.sum(p, axis=1, keepdims=True)
                o = lax.dot(p.astype(jnp.bfloat16), v_ref[b, :, h, :],
                            preferred_element_type=jnp.float32)
                acc_ref[r0:r0 + Sq, c0:c0 + Dh] = o
                acc_ref[r0:r0 + Sq, D + h:D + h + 1] = lvec
            acc_ref[r0:r0 + Sq, D + Hq:] = jnp.zeros(
                (Sq, EXT - Hq), jnp.float32)

        comm_ref[0] = acc_ref[...].astype(jnp.bfloat16)

        for hop in range(N_DEV - 1):
            s_slot = hop % 2
            r_slot = (hop + 1) % 2
            rdma = pltpu.make_async_remote_copy(
                src_ref=comm_ref.at[s_slot],
                dst_ref=comm_ref.at[r_slot],
                send_sem=send_sems.at[s_slot],
                recv_sem=recv_sems.at[r_slot],
                device_id=(right,),
                device_id_type=pl.DeviceIdType.MESH,
            )
            if hop >= 1:
                pl.semaphore_wait(credit_sem, 1)
            rdma.start()
            rdma.wait()
            acc_ref[...] += comm_ref[r_slot].astype(jnp.float32)
            if hop < N_DEV - 2:
                pl.semaphore_signal(credit_sem, inc=1, device_id=(left,),
                                    device_id_type=pl.DeviceIdType.MESH)

        for b in range(B):
            r0 = b * Sq
            for h in range(Hq):
                c0 = h * Dh
                lcol = acc_ref[r0:r0 + Sq, D + h:D + h + 1]
                attn_ref[r0:r0 + Sq, c0:c0 + Dh] = (
                    acc_ref[r0:r0 + Sq, c0:c0 + Dh] / lcol
                ).astype(jnp.bfloat16)
        res = lax.dot(attn_ref[...], wo_ref[...],
                      preferred_element_type=jnp.float32)
        out_ref[...] = res.reshape(B, Sq, D)

    return pl.pallas_call(
        body,
        out_shape=jax.ShapeDtypeStruct((B, Sq, D), jnp.float32),
        in_specs=[pl.BlockSpec(memory_space=pltpu.VMEM)] * 5,
        out_specs=pl.BlockSpec(memory_space=pltpu.VMEM),
        scratch_shapes=[
            pltpu.VMEM((B * Sq, D), jnp.bfloat16),
            pltpu.VMEM((B * Sq, W), jnp.float32),
            pltpu.VMEM((2, B * Sq, W), jnp.bfloat16),
            pltpu.VMEM((B * Sq, D), jnp.bfloat16),
            pltpu.SemaphoreType.DMA((2,)),
            pltpu.SemaphoreType.DMA((2,)),
            pltpu.SemaphoreType.REGULAR,
        ],
        compiler_params=pltpu.CompilerParams(collective_id=0),
    )(xb, wqb, wob, kb, vb)


# device time: 116626 ns/iter; 4.5970x vs baseline; 4.5970x over previous
import jax
import jax.numpy as jnp
from jax import lax
from jax.experimental import pallas as pl
from jax.experimental.pallas import tpu as pltpu

N_DEV = 16
B, Sq, Hq, Dh = 4, 256, 8, 128
D = Hq * Dh
SCALE = 0.08838834764831843
EXT = 128
W = D + EXT
R = (B * Sq) // N_DEV


def kernel(x, Wq, Wo, K_ext, V_ext):
    xb = x.astype(jnp.bfloat16)
    wqb = Wq.astype(jnp.bfloat16)
    wob = Wo.astype(jnp.bfloat16)
    kb = K_ext.astype(jnp.bfloat16)
    vb = V_ext.astype(jnp.bfloat16)

    def body(x_ref, wq_ref, wo_ref, k_ref, v_ref, out_ref,
             q_ref, acc_ref, st_ref, rs_ref, fin_ref, attn_ref, og_ref,
             rs_send, rs_recv, ag_send, ag_recv):
        my = lax.axis_index("i")
        my_row = my * R

        def rs_rdma(c):
            return pltpu.make_async_remote_copy(
                src_ref=st_ref.at[c],
                dst_ref=rs_ref.at[my],
                send_sem=rs_send.at[c],
                recv_sem=rs_recv.at[my],
                device_id=(c,),
                device_id_type=pl.DeviceIdType.MESH,
            )

        def rs_recv_rdma(s):
            return pltpu.make_async_remote_copy(
                src_ref=st_ref.at[s],
                dst_ref=rs_ref.at[s],
                send_sem=rs_send.at[s],
                recv_sem=rs_recv.at[s],
                device_id=(s,),
                device_id_type=pl.DeviceIdType.MESH,
            )

        def ag_rdma(t):
            return pltpu.make_async_remote_copy(
                src_ref=og_ref.at[pl.ds(my_row, R)],
                dst_ref=og_ref.at[pl.ds(my_row, R)],
                send_sem=ag_send.at[t],
                recv_sem=ag_recv.at[my],
                device_id=(t,),
                device_id_type=pl.DeviceIdType.MESH,
            )

        def ag_recv_rdma(s):
            return pltpu.make_async_remote_copy(
                src_ref=og_ref.at[pl.ds(s * R, R)],
                dst_ref=og_ref.at[pl.ds(s * R, R)],
                send_sem=ag_send.at[s],
                recv_sem=ag_recv.at[s],
                device_id=(s,),
                device_id_type=pl.DeviceIdType.MESH,
            )

        xm = x_ref[...].reshape(B * Sq, D)
        q_ref[...] = (lax.dot(xm, wq_ref[...],
                              preferred_element_type=jnp.float32)
                      * SCALE).astype(jnp.bfloat16)

        for b in range(B):
            r0 = b * Sq
            for h in range(Hq):
                c0 = h * Dh
                qbh = q_ref[r0:r0 + Sq, c0:c0 + Dh]
                kbh = k_ref[b, :, h, :]
                s = lax.dot_general(qbh, kbh, (((1,), (1,)), ((), ())),
                                    preferred_element_type=jnp.float32)
                p = jnp.exp(s)
                lvec = jnp.sum(p, axis=1, keepdims=True)
                o = lax.dot(p.astype(jnp.bfloat16), v_ref[b, :, h, :],
                            preferred_element_type=jnp.float32)
                acc_ref[r0:r0 + Sq, c0:c0 + Dh] = o
                acc_ref[r0:r0 + Sq, D + h:D + h + 1] = lvec
            acc_ref[r0:r0 + Sq, D + Hq:] = jnp.zeros(
                (Sq, EXT - Hq), jnp.float32)
            for c in range(4 * b, 4 * b + 4):
                @pl.when(c != my)
                def _(c=c):
                    st_ref[c] = acc_ref[c * R:(c + 1) * R, :].astype(
                        jnp.bfloat16)
                    rs_rdma(c).start()

        fin_ref[...] = acc_ref[pl.ds(my_row, R), :]
        for s in range(N_DEV):
            @pl.when(s != my)
            def _(s=s):
                rs_recv_rdma(s).wait_recv()
                fin_ref[...] += rs_ref[s].astype(jnp.float32)

        for h in range(Hq):
            c0 = h * Dh
            attn_ref[:, c0:c0 + Dh] = (
                fin_ref[:, c0:c0 + Dh] / fin_ref[:, D + h:D + h + 1]
            ).astype(jnp.bfloat16)
        outc = lax.dot(attn_ref[...], wo_ref[...],
                       preferred_element_type=jnp.float32)
        og_ref[pl.ds(my_row, R), :] = outc.astype(jnp.bfloat16)

        for t in range(N_DEV):
            @pl.when(t != my)
            def _(t=t):
                ag_rdma(t).start()
        for s in range(N_DEV):
            @pl.when(s != my)
            def _(s=s):
                ag_recv_rdma(s).wait_recv()

        for c in range(N_DEV):
            @pl.when(c != my)
            def _(c=c):
                rs_rdma(c).wait_send()
                ag_rdma(c).wait_send()

        out_ref[...] = og_ref[...].astype(jnp.float32).reshape(B, Sq, D)

    return pl.pallas_call(
        body,
        out_shape=jax.ShapeDtypeStruct((B, Sq, D), jnp.float32),
        in_specs=[pl.BlockSpec(memory_space=pltpu.VMEM)] * 5,
        out_specs=pl.BlockSpec(memory_space=pltpu.VMEM),
        scratch_shapes=[
            pltpu.VMEM((B * Sq, D), jnp.bfloat16),
            pltpu.VMEM((B * Sq, W), jnp.float32),
            pltpu.VMEM((N_DEV, R, W), jnp.bfloat16),
            pltpu.VMEM((N_DEV, R, W), jnp.bfloat16),
            pltpu.VMEM((R, W), jnp.float32),
            pltpu.VMEM((R, D), jnp.bfloat16),
            pltpu.VMEM((B * Sq, D), jnp.bfloat16),
            pltpu.SemaphoreType.DMA((N_DEV,)),
            pltpu.SemaphoreType.DMA((N_DEV,)),
            pltpu.SemaphoreType.DMA((N_DEV,)),
            pltpu.SemaphoreType.DMA((N_DEV,)),
        ],
    )(xb, wqb, wob, kb, vb)
